# Initial kernel scaffold; baseline (speedup 1.0000x reference)
import jax
import jax.numpy as jnp
from jax.experimental import pallas as pl


def kernel(x, edge_index, batch, global_features, Wl0, Wr0, b0, g0, be0, Wl, Wr, bS, gS, beS, Wm0, bm0, gm0, bem0, Wm, bm, gm, bem, Wf, bf):
    raise NotImplementedError("write your pallas kernel here")


# speedup vs baseline: 4.7809x; 4.7809x over previous
import functools

import jax
import jax.numpy as jnp
from jax import lax
from jax.experimental import pallas as pl
from jax.experimental.pallas import tpu as pltpu
from jax.experimental.pallas import tpu_sc as plsc

N = 10000
E = 320000
D = 128
H = 64
G = 24
B = 64
LC = 32
SL = 6
LL = 3

NW = 32
CHUNK = 128
NCH = 80
EPT = NCH * CHUNK
E_PAD = EPT * NW
N_ACC = 16000
RPT = N_ACC // 16
BLK = 1000
NBLK = N // BLK
ACC1_OFF = N_ACC // BLK

_f32 = jnp.float32


def _count_body(dst_hbm, out_hbm, dst_v, ones_v, zb_v, acc, sem):
    c = lax.axis_index("c")
    s = lax.axis_index("s")
    w = c * 16 + s
    pltpu.sync_copy(dst_hbm.at[w], dst_v)

    one16 = jnp.ones((16,), _f32)
    zero16 = jnp.zeros((16,), _f32)

    def _init_row(i, _):
        ones_v[i, :] = one16
        zb_v[i, :] = zero16
        return 0
    lax.fori_loop(0, CHUNK, _init_row, 0)

    base = s * RPT
    for k in range(RPT // CHUNK):
        pltpu.sync_copy(zb_v, acc.at[pl.ds(base + k * CHUNK, CHUNK)])
    if RPT % CHUNK:
        pltpu.sync_copy(zb_v.at[pl.ds(0, RPT % CHUNK)],
                        acc.at[pl.ds(base + (RPT // CHUNK) * CHUNK,
                                     RPT % CHUNK)])
    plsc.subcore_barrier()

    def _group(g, _):
        descs = []
        for b in range(4):
            j = g * 4 + b
            descs.append(
                pltpu.async_copy(ones_v, acc.at[dst_v.at[j]], sem, add=True))
        for d in descs:
            d.wait()
        return 0
    lax.fori_loop(0, NCH // 4, _group, 0)

    plsc.subcore_barrier()
    pltpu.sync_copy(acc.at[pl.ds(s * RPT, RPT)],
                    out_hbm.at[pl.ds(c * N_ACC + s * RPT, RPT)])


_count_call = functools.partial(
    pl.kernel,
    out_type=jax.ShapeDtypeStruct((2 * N_ACC, 16), _f32),
    mesh=plsc.VectorSubcoreMesh(core_axis_name="c", subcore_axis_name="s"),
    scratch_types=[
        pltpu.VMEM((NCH, CHUNK), jnp.int32),
        pltpu.VMEM((CHUNK, 16), _f32),
        pltpu.VMEM((CHUNK, 16), _f32),
        pltpu.VMEM_SHARED((N_ACC, 16), _f32),
        pltpu.SemaphoreType.DMA,
    ],
    compiler_params=pltpu.CompilerParams(use_tc_tiling_on_sc=False),
)(_count_body)


def _edge_body(y_hbm, src_hbm, dst_hbm, out_hbm,
               src_v, dst_v, rows_v, zb_v, acc, sem_g, sem_s):
    c = lax.axis_index("c")
    s = lax.axis_index("s")
    w = c * 16 + s
    pltpu.sync_copy(src_hbm.at[w], src_v)
    pltpu.sync_copy(dst_hbm.at[w], dst_v)

    zero16 = jnp.zeros((16,), _f32)

    def _zrow(i, _):
        for l in range(4):
            zb_v[i, pl.ds(l * 16, 16)] = zero16
        return 0
    lax.fori_loop(0, CHUNK, _zrow, 0)

    base = s * RPT
    for k in range(RPT // CHUNK):
        pltpu.sync_copy(zb_v, acc.at[pl.ds(base + k * CHUNK, CHUNK)])
    if RPT % CHUNK:
        pltpu.sync_copy(zb_v.at[pl.ds(0, RPT % CHUNK)],
                        acc.at[pl.ds(base + (RPT // CHUNK) * CHUNK,
                                     RPT % CHUNK)])
    plsc.subcore_barrier()

    def _group(g, _):
        gds = []
        for b in range(4):
            j = g * 4 + b
            gds.append(
                pltpu.async_copy(y_hbm.at[src_v.at[j]], rows_v.at[b], sem_g))
        for d in gds:
            d.wait()
        sds = []
        for b in range(4):
            j = g * 4 + b
            sds.append(
                pltpu.async_copy(rows_v.at[b], acc.at[dst_v.at[j]], sem_s,
                                 add=True))
        for d in sds:
            d.wait()
        return 0
    lax.fori_loop(0, NCH // 4, _group, 0)

    plsc.subcore_barrier()
    pltpu.sync_copy(acc.at[pl.ds(s * RPT, RPT)],
                    out_hbm.at[pl.ds(c * N_ACC + s * RPT, RPT)])


_edge_call = functools.partial(
    pl.kernel,
    out_type=jax.ShapeDtypeStruct((2 * N_ACC, H), _f32),
    mesh=plsc.VectorSubcoreMesh(core_axis_name="c", subcore_axis_name="s"),
    scratch_types=[
        pltpu.VMEM((NCH, CHUNK), jnp.int32),
        pltpu.VMEM((NCH, CHUNK), jnp.int32),
        pltpu.VMEM((4, CHUNK, H), _f32),
        pltpu.VMEM((CHUNK, H), _f32),
        pltpu.VMEM_SHARED((N_ACC, H), _f32),
        pltpu.SemaphoreType.DMA,
        pltpu.SemaphoreType.DMA,
    ],
    compiler_params=pltpu.CompilerParams(use_tc_tiling_on_sc=False),
)(_edge_body)


def _ln(u, g, be):
    mu = jnp.mean(u, axis=-1, keepdims=True)
    var = jnp.mean((u - mu) ** 2, axis=-1, keepdims=True)
    return (u - mu) * lax.rsqrt(var + 1e-5) * g + be


def _gelu(x):
    return 0.5 * x * (1.0 + lax.erf(x * (2.0 ** -0.5)))


def _pre_body(x_ref, wl_ref, wr_ref, cnt0_ref, cnt1_ref,
              y_ref, z_ref, inv_ref):
    xb = x_ref[...]
    y_ref[...] = jnp.dot(xb, wl_ref[...], preferred_element_type=_f32)
    z_ref[...] = jnp.dot(xb, wr_ref[...], preferred_element_type=_f32)
    cnt = jnp.max(cnt0_ref[...] + cnt1_ref[...], axis=-1, keepdims=True)
    inv_ref[...] = jnp.broadcast_to(1.0 / jnp.maximum(cnt, 1.0), (BLK, H))


def _pre_call(x, wl0, wr0, cnt2):
    return pl.pallas_call(
        _pre_body,
        grid=(NBLK,),
        in_specs=[
            pl.BlockSpec((BLK, D), lambda b: (b, 0)),
            pl.BlockSpec((D, H), lambda b: (0, 0)),
            pl.BlockSpec((D, H), lambda b: (0, 0)),
            pl.BlockSpec((BLK, 16), lambda b: (b, 0)),
            pl.BlockSpec((BLK, 16), lambda b: (b + ACC1_OFF, 0)),
        ],
        out_specs=[
            pl.BlockSpec((BLK, H), lambda b: (b, 0)),
            pl.BlockSpec((BLK, H), lambda b: (b, 0)),
            pl.BlockSpec((BLK, H), lambda b: (b, 0)),
        ],
        out_shape=[
            jax.ShapeDtypeStruct((N, H), _f32),
            jax.ShapeDtypeStruct((N, H), _f32),
            jax.ShapeDtypeStruct((N, H), _f32),
        ],
    )(x, wl0, wr0, cnt2, cnt2)


def _mid_body(residual, acc0_ref, acc1_ref, z_ref, h_ref, inv_ref,
              bias_ref, g_ref, be_ref, wln_ref, wrn_ref,
              h_out, y_out, z_out):
    mean = (acc0_ref[...] + acc1_ref[...]) * inv_ref[...]
    pre = mean + z_ref[...] + bias_ref[...]
    f = _ln(_gelu(pre), g_ref[...], be_ref[...])
    hn = f + h_ref[...] if residual else f
    h_out[...] = hn
    y_out[...] = jnp.dot(hn, wln_ref[...], preferred_element_type=_f32)
    z_out[...] = jnp.dot(hn, wrn_ref[...], preferred_element_type=_f32)


def _mid_call(residual, acc, z, h, inv, bias, g, be, wln, wrn):
    return pl.pallas_call(
        functools.partial(_mid_body, residual),
        grid=(NBLK,),
        in_specs=[
            pl.BlockSpec((BLK, H), lambda b: (b, 0)),
            pl.BlockSpec((BLK, H), lambda b: (b + ACC1_OFF, 0)),
            pl.BlockSpec((BLK, H), lambda b: (b, 0)),
            pl.BlockSpec((BLK, H), lambda b: (b, 0)),
            pl.BlockSpec((BLK, H), lambda b: (b, 0)),
            pl.BlockSpec((1, H), lambda b: (0, 0)),
            pl.BlockSpec((1, H), lambda b: (0, 0)),
            pl.BlockSpec((1, H), lambda b: (0, 0)),
            pl.BlockSpec((H, H), lambda b: (0, 0)),
            pl.BlockSpec((H, H), lambda b: (0, 0)),
        ],
        out_specs=[
            pl.BlockSpec((BLK, H), lambda b: (b, 0)),
            pl.BlockSpec((BLK, H), lambda b: (b, 0)),
            pl.BlockSpec((BLK, H), lambda b: (b, 0)),
        ],
        out_shape=[
            jax.ShapeDtypeStruct((N, H), _f32),
            jax.ShapeDtypeStruct((N, H), _f32),
            jax.ShapeDtypeStruct((N, H), _f32),
        ],
    )(acc, acc, z, h, inv, bias, g, be, wln, wrn)


def _fin_body(acc0_ref, acc1_ref, z_ref, h_ref, inv_ref,
              bias_ref, g_ref, be_ref, batch_ref, gf_ref,
              wm0_ref, bm0_ref, gm0_ref, bem0_ref,
              wm_ref, bm_ref, gm_ref, bem_ref, wf_ref, bf_ref,
              out_ref, pool_acc):
    bidx = pl.program_id(0)
    mean = (acc0_ref[...] + acc1_ref[...]) * inv_ref[...]
    pre = mean + z_ref[...] + bias_ref[...]
    f = _ln(_gelu(pre), g_ref[...], be_ref[...])
    hn = f + h_ref[...]

    lanes = lax.broadcasted_iota(jnp.int32, (1, B), 1)
    onehot = (batch_ref[...] == lanes).astype(_f32)
    pb = lax.dot_general(onehot, hn, (((0,), (0,)), ((), ())),
                         preferred_element_type=_f32)

    @pl.when(bidx == 0)
    def _():
        pool_acc[...] = pb

    @pl.when(bidx > 0)
    def _():
        pool_acc[...] = pool_acc[...] + pb

    @pl.when(bidx == NBLK - 1)
    def _():
        pool = pool_acc[...]
        m = (jnp.dot(pool, wm0_ref[0:H, :], preferred_element_type=_f32)
             + jnp.dot(gf_ref[...], wm0_ref[H:H + G, :],
                       preferred_element_type=_f32)
             + bm0_ref[...])
        m = _ln(_gelu(m), gm0_ref[...], bem0_ref[...])
        for i in range(LL):
            t = jnp.dot(m, wm_ref[i], preferred_element_type=_f32) + bm_ref[i]
            m = _ln(_gelu(t), gm_ref[i], bem_ref[i]) + m
        out_ref[...] = jnp.dot(m, wf_ref[...],
                               preferred_element_type=_f32) + bf_ref[...]


def _fin_call(acc, z, h, inv, bias, g, be, batch2, gf,
              wm0, bm0, gm0, bem0, wm, bm, gm, bem, wf, bf):
    return pl.pallas_call(
        _fin_body,
        grid=(NBLK,),
        in_specs=[
            pl.BlockSpec((BLK, H), lambda b: (b, 0)),
            pl.BlockSpec((BLK, H), lambda b: (b + ACC1_OFF, 0)),
            pl.BlockSpec((BLK, H), lambda b: (b, 0)),
            pl.BlockSpec((BLK, H), lambda b: (b, 0)),
            pl.BlockSpec((BLK, H), lambda b: (b, 0)),
            pl.BlockSpec((1, H), lambda b: (0, 0)),
            pl.BlockSpec((1, H), lambda b: (0, 0)),
            pl.BlockSpec((1, H), lambda b: (0, 0)),
            pl.BlockSpec((BLK, 1), lambda b: (b, 0)),
            pl.BlockSpec((B, G), lambda b: (0, 0)),
            pl.BlockSpec((H + G, LC), lambda b: (0, 0)),
            pl.BlockSpec((1, LC), lambda b: (0, 0)),
            pl.BlockSpec((1, LC), lambda b: (0, 0)),
            pl.BlockSpec((1, LC), lambda b: (0, 0)),
            pl.BlockSpec((LL, LC, LC), lambda b: (0, 0, 0)),
            pl.BlockSpec((LL, 1, LC), lambda b: (0, 0, 0)),
            pl.BlockSpec((LL, 1, LC), lambda b: (0, 0, 0)),
            pl.BlockSpec((LL, 1, LC), lambda b: (0, 0, 0)),
            pl.BlockSpec((LC, 1), lambda b: (0, 0)),
            pl.BlockSpec((1, 1), lambda b: (0, 0)),
        ],
        out_specs=pl.BlockSpec((B, 1), lambda b: (0, 0)),
        out_shape=jax.ShapeDtypeStruct((B, 1), _f32),
        scratch_shapes=[pltpu.VMEM((B, H), _f32)],
    )(acc, acc, z, h, inv, bias, g, be, batch2, gf,
      wm0, bm0, gm0, bem0, wm, bm, gm, bem, wf, bf)


def kernel(x, edge_index, batch, global_features, Wl0, Wr0, b0, g0, be0,
           Wl, Wr, bS, gS, beS, Wm0, bm0, gm0, bem0, Wm, bm, gm, bem,
           Wf, bf):
    src = edge_index[0]
    dst = edge_index[1]
    pad = E_PAD - E
    src_p = jnp.concatenate([src, jnp.zeros((pad,), jnp.int32)])
    dst_p = jnp.concatenate([dst, jnp.full((pad,), N, jnp.int32)])
    src_p = src_p.reshape(NW, NCH, CHUNK)
    dst_p = dst_p.reshape(NW, NCH, CHUNK)
    batch2 = batch.reshape(N, 1)

    cnt2 = _count_call(dst_p)
    y, z, inv = _pre_call(x, Wl0, Wr0, cnt2)

    h = jnp.zeros((N, H), _f32)
    bias, g, be = b0.reshape(1, H), g0.reshape(1, H), be0.reshape(1, H)
    for i in range(SL):
        acc = _edge_call(y, src_p, dst_p)
        h, y, z = _mid_call(i > 0, acc, z, h, inv, bias, g, be, Wl[i], Wr[i])
        bias, g, be = (bS[i].reshape(1, H), gS[i].reshape(1, H),
                       beS[i].reshape(1, H))

    acc = _edge_call(y, src_p, dst_p)
    out = _fin_call(acc, z, h, inv, bias, g, be, batch2, global_features,
                    Wm0, bm0.reshape(1, LC), gm0.reshape(1, LC),
                    bem0.reshape(1, LC), Wm, bm.reshape(LL, 1, LC),
                    gm.reshape(LL, 1, LC), bem.reshape(LL, 1, LC),
                    Wf, bf.reshape(1, 1))
    return out
